# baseline (speedup 1.0000x reference)
import functools

import numpy as np
import jax
import jax.numpy as jnp
from jax.experimental import pallas as pl
from jax.experimental.pallas import tpu as pltpu

_KSIZE = 3
_KK = _KSIZE * _KSIZE
_EPS = 1e-5


def _rot_maps(k):
    i, j = np.meshgrid(np.arange(k), np.arange(k), indexing="ij")
    return np.stack([
        (i * k + j).ravel(),
        (j * k + (k - 1 - i)).ravel(),
        ((k - 1 - i) * k + (k - 1 - j)).ravel(),
        ((k - 1 - j) * k + i).ravel(),
    ], axis=0)


def _conv2_row_index(c_out):
    rot = _rot_maps(_KSIZE)
    taps, chans = np.meshgrid(np.arange(_KK), np.arange(c_out), indexing="ij")
    r = chans % 4
    q = (chans // 4) * _KK + rot[r, taps]
    return q.ravel(), r.ravel()


def _prep_kernel(w1r_ref, wvr_ref, w2_ref, b2_ref, qf_ref, rf_ref,
                 w1d_ref, wvd_ref, w2p_ref):
    def c4_expand(wr):
        o, n = wr.shape
        kk = jax.lax.broadcasted_iota(jnp.int32, (n, n), 0)
        nn = jax.lax.broadcasted_iota(jnp.int32, (n, n), 1)
        parts = []
        for s in range(4):
            src = nn - nn % 4 + (nn % 4 - s) % 4
            perm = (kk == src).astype(jnp.float32)
            parts.append(jnp.dot(wr[...], perm,
                                 preferred_element_type=jnp.float32))
        return jnp.stack(parts, axis=1).reshape(4 * o, n)

    w1d_ref[...] = c4_expand(w1r_ref).astype(w1d_ref.dtype)
    wvd_ref[...] = c4_expand(wvr_ref).astype(wvd_ref.dtype)

    nrow = qf_ref.shape[0]
    nq = w2_ref.shape[0]
    mid = w2_ref.shape[1]
    qcol = jax.lax.broadcasted_iota(jnp.int32, (nrow, nq), 1)
    onehot = (qcol == qf_ref[...]).astype(jnp.float32)
    rows4 = jnp.dot(onehot, w2_ref[...], preferred_element_type=jnp.float32)
    b2p = jnp.dot(onehot, b2_ref[...], preferred_element_type=jnp.float32)
    mm = jax.lax.broadcasted_iota(jnp.int32, (mid, 4 * mid), 0)
    nn = jax.lax.broadcasted_iota(jnp.int32, (mid, 4 * mid), 1)
    widen = (nn // 4 == mm).astype(jnp.float32)
    wide = jnp.dot(rows4, widen, preferred_element_type=jnp.float32)
    tcol = jax.lax.broadcasted_iota(jnp.int32, (nrow, 4 * mid), 1) % 4
    w2p = jnp.where(tcol == rf_ref[...], wide, 0.0)
    bcol = jax.lax.broadcasted_iota(jnp.int32, (nrow, 4 * mid), 1)
    bias_block = jnp.where(bcol == 0, b2p, 0.0)
    w2p_ref[...] = jnp.concatenate([w2p, bias_block],
                                   axis=1).astype(w2p_ref.dtype)


def _expand_weights(w1, w2, b2, wv, dt):
    mid4 = 4 * w1.shape[0]
    c_out = 4 * wv.shape[0]
    kc = _KK * c_out
    qf, rf = _conv2_row_index(c_out)
    return pl.pallas_call(
        _prep_kernel,
        out_shape=(jax.ShapeDtypeStruct((mid4, 4 * w1.shape[1]), dt),
                   jax.ShapeDtypeStruct((c_out, 4 * wv.shape[1]), dt),
                   jax.ShapeDtypeStruct((kc, 2 * mid4), jnp.bfloat16)),
    )(w1.reshape(w1.shape[0], -1), wv.reshape(wv.shape[0], -1),
      w2, b2.reshape(-1, 1),
      jnp.asarray(qf.reshape(-1, 1), jnp.int32),
      jnp.asarray(rf.reshape(-1, 1), jnp.int32))


def _fused_kernel(x_ref, w1_ref, gam_ref, bet_ref, w2p_ref, wv_ref,
                  o_ref, *, img_w, eps, g_batch):
    pad = 2 * img_w
    s = x_ref.shape[-1]
    mid4 = w1_ref.shape[0]
    c_out = wv_ref.shape[0]
    rr = jax.lax.broadcasted_iota(jnp.int32, (mid4, mid4), 0)
    cc = jax.lax.broadcasted_iota(jnp.int32, (mid4, mid4), 1)
    gmask = (rr // 4 == cc // 4).astype(jnp.float32)
    onesrow = (jax.lax.broadcasted_iota(jnp.int32, (mid4, s), 0) == 0
               ).astype(jnp.float32).astype(jnp.bfloat16)
    jcol = jax.lax.broadcasted_iota(jnp.int32, (1, s + 2 * pad), 1) % img_w
    mask_l = (jcol <= img_w - 2).astype(jnp.float32)
    mask_r = (jcol >= 1).astype(jnp.float32)
    n = jnp.float32(4 * s)

    for g in range(g_batch):
        x = x_ref[g]

        y = jnp.dot(w1_ref[...], x, preferred_element_type=jnp.float32)

        ysum = jnp.sum(y, axis=1, keepdims=True)
        y2sum = jnp.sum(y * y, axis=1, keepdims=True)
        gmean = jnp.dot(gmask, ysum, preferred_element_type=jnp.float32) / n
        gm2 = jnp.dot(gmask, y2sum, preferred_element_type=jnp.float32) / n
        inv = jax.lax.rsqrt(gm2 - gmean * gmean + eps)
        scale = inv * gam_ref[...]
        shift = bet_ref[...] - gmean * scale
        yr = jnp.maximum(y * scale + shift, 0.0)

        yrb = jnp.concatenate([yr.astype(jnp.bfloat16), onesrow], axis=0)
        wpred = jnp.dot(w2p_ref[...], yrb,
                        preferred_element_type=jnp.float32
                        ).astype(jnp.bfloat16)

        v = jnp.dot(wv_ref[...].astype(jnp.bfloat16),
                    x.astype(jnp.bfloat16),
                    preferred_element_type=jnp.float32
                    ).astype(jnp.bfloat16)
        zpad = jnp.zeros((c_out, pad), jnp.bfloat16)
        vp = jnp.concatenate([zpad, v, zpad], axis=1)

        base_of = {0: vp * mask_l.astype(jnp.bfloat16), 1: vp,
                   2: vp * mask_r.astype(jnp.bfloat16)}

        acc = None
        for p in range(_KK):
            di, dj = divmod(p, _KSIZE)
            start = pad + (di - 1) * img_w + (dj - 1)
            vs = base_of[dj][:, start:start + s]
            term = wpred[p * c_out:(p + 1) * c_out, :] * vs
            acc = term if acc is None else acc + term
        o_ref[g] = acc.astype(o_ref.dtype)


def kernel(x, w1, gn_gamma, gn_beta, w2, b2, wv):
    bsz, c, h, w = x.shape
    s = h * w
    mid4 = 4 * w1.shape[0]
    c_out = 4 * wv.shape[0]
    dt = x.dtype

    x_flat = x.reshape(bsz, c, s)
    w1d, wvd, w2p = _expand_weights(w1, w2, b2, wv, dt)
    gam = gn_gamma.reshape(mid4, 1).astype(dt)
    bet = gn_beta.reshape(mid4, 1).astype(dt)

    g_batch = 2 if bsz % 2 == 0 else 1
    out = pl.pallas_call(
        functools.partial(_fused_kernel, img_w=w, eps=_EPS, g_batch=g_batch),
        out_shape=jax.ShapeDtypeStruct((bsz, c_out, s), dt),
        grid=(bsz // g_batch,),
        in_specs=[
            pl.BlockSpec((g_batch, c, s), lambda b: (b, 0, 0)),
            pl.BlockSpec((mid4, c), lambda b: (0, 0)),
            pl.BlockSpec((mid4, 1), lambda b: (0, 0)),
            pl.BlockSpec((mid4, 1), lambda b: (0, 0)),
            pl.BlockSpec((_KK * c_out, 2 * mid4), lambda b: (0, 0)),
            pl.BlockSpec((c_out, c), lambda b: (0, 0)),
        ],
        out_specs=pl.BlockSpec((g_batch, c_out, s), lambda b: (b, 0, 0)),
        compiler_params=pltpu.CompilerParams(
            dimension_semantics=("parallel",)),
    )(x_flat, w1d, gam, bet, w2p, wvd)
    return out.reshape(bsz, c_out, h, w)


# speedup vs baseline: 1.0144x; 1.0144x over previous
import functools

import numpy as np
import jax
import jax.numpy as jnp
from jax.experimental import pallas as pl
from jax.experimental.pallas import tpu as pltpu

_KSIZE = 3
_KK = _KSIZE * _KSIZE
_EPS = 1e-5


def _rot_maps(k):
    i, j = np.meshgrid(np.arange(k), np.arange(k), indexing="ij")
    return np.stack([
        (i * k + j).ravel(),
        (j * k + (k - 1 - i)).ravel(),
        ((k - 1 - i) * k + (k - 1 - j)).ravel(),
        ((k - 1 - j) * k + i).ravel(),
    ], axis=0)


def _conv2_row_index(c_out):
    rot = _rot_maps(_KSIZE)
    taps, chans = np.meshgrid(np.arange(_KK), np.arange(c_out), indexing="ij")
    r = chans % 4
    q = (chans // 4) * _KK + rot[r, taps]
    return q.ravel(), r.ravel()


def _prep_kernel(w1r_ref, wvr_ref, w2_ref, b2_ref, qf_ref, rf_ref,
                 w1d_ref, wvd_ref, w2p_ref):
    def c4_expand(wr):
        o, n = wr.shape
        kk = jax.lax.broadcasted_iota(jnp.int32, (n, n), 0)
        nn = jax.lax.broadcasted_iota(jnp.int32, (n, n), 1)
        parts = []
        for s in range(4):
            src = nn - nn % 4 + (nn % 4 - s) % 4
            perm = (kk == src).astype(jnp.float32)
            parts.append(jnp.dot(wr[...], perm,
                                 preferred_element_type=jnp.float32))
        return jnp.stack(parts, axis=1).reshape(4 * o, n)

    w1d_ref[...] = c4_expand(w1r_ref).astype(w1d_ref.dtype)
    wvd_ref[...] = c4_expand(wvr_ref).astype(wvd_ref.dtype)

    nrow = qf_ref.shape[0]
    nq = w2_ref.shape[0]
    mid = w2_ref.shape[1]
    qcol = jax.lax.broadcasted_iota(jnp.int32, (nrow, nq), 1)
    onehot = (qcol == qf_ref[...]).astype(jnp.float32)
    rows4 = jnp.dot(onehot, w2_ref[...], preferred_element_type=jnp.float32)
    b2p = jnp.dot(onehot, b2_ref[...], preferred_element_type=jnp.float32)
    mm = jax.lax.broadcasted_iota(jnp.int32, (mid, 4 * mid), 0)
    nn = jax.lax.broadcasted_iota(jnp.int32, (mid, 4 * mid), 1)
    widen = (nn // 4 == mm).astype(jnp.float32)
    wide = jnp.dot(rows4, widen, preferred_element_type=jnp.float32)
    tcol = jax.lax.broadcasted_iota(jnp.int32, (nrow, 4 * mid), 1) % 4
    w2p = jnp.where(tcol == rf_ref[...], wide, 0.0)
    bcol = jax.lax.broadcasted_iota(jnp.int32, (nrow, 4 * mid), 1)
    bias_block = jnp.where(bcol == 0, b2p, 0.0)
    w2p_ref[...] = jnp.concatenate([w2p, bias_block],
                                   axis=1).astype(w2p_ref.dtype)


def _expand_weights(w1, w2, b2, wv, dt):
    mid4 = 4 * w1.shape[0]
    c_out = 4 * wv.shape[0]
    kc = _KK * c_out
    qf, rf = _conv2_row_index(c_out)
    return pl.pallas_call(
        _prep_kernel,
        out_shape=(jax.ShapeDtypeStruct((mid4, 4 * w1.shape[1]), dt),
                   jax.ShapeDtypeStruct((c_out, 4 * wv.shape[1]), dt),
                   jax.ShapeDtypeStruct((kc, 2 * mid4), jnp.bfloat16)),
    )(w1.reshape(w1.shape[0], -1), wv.reshape(wv.shape[0], -1),
      w2, b2.reshape(-1, 1),
      jnp.asarray(qf.reshape(-1, 1), jnp.int32),
      jnp.asarray(rf.reshape(-1, 1), jnp.int32))


def _fused_kernel(x_ref, w1_ref, gam_ref, bet_ref, w2p_ref, wv_ref,
                  o_ref, *, img_w, eps, g_batch):
    pad = 2 * img_w
    s = x_ref.shape[-1]
    mid4 = w1_ref.shape[0]
    c_out = wv_ref.shape[0]
    rr = jax.lax.broadcasted_iota(jnp.int32, (mid4, mid4), 0)
    cc = jax.lax.broadcasted_iota(jnp.int32, (mid4, mid4), 1)
    gmask = (rr // 4 == cc // 4).astype(jnp.float32)
    onesrow = (jax.lax.broadcasted_iota(jnp.int32, (mid4, s), 0) == 0
               ).astype(jnp.float32).astype(jnp.bfloat16)
    jcol = jax.lax.broadcasted_iota(jnp.int32, (1, s + 2 * pad), 1) % img_w
    mask_l = (jcol <= img_w - 2).astype(jnp.float32)
    mask_r = (jcol >= 1).astype(jnp.float32)
    n = jnp.float32(4 * s)

    for g in range(g_batch):
        x = x_ref[g]

        y = jnp.dot(w1_ref[...], x, preferred_element_type=jnp.float32)

        ysum = jnp.sum(y, axis=1, keepdims=True)
        y2sum = jnp.sum(y * y, axis=1, keepdims=True)
        gmean = jnp.dot(gmask, ysum, preferred_element_type=jnp.float32) / n
        gm2 = jnp.dot(gmask, y2sum, preferred_element_type=jnp.float32) / n
        inv = jax.lax.rsqrt(gm2 - gmean * gmean + eps)
        scale = inv * gam_ref[...]
        shift = bet_ref[...] - gmean * scale
        yr = jnp.maximum(y * scale + shift, 0.0)

        yrb = jnp.concatenate([yr.astype(jnp.bfloat16), onesrow], axis=0)
        wpred = jnp.dot(w2p_ref[...], yrb,
                        preferred_element_type=jnp.float32
                        ).astype(jnp.bfloat16)

        v = jnp.dot(wv_ref[...].astype(jnp.bfloat16),
                    x.astype(jnp.bfloat16),
                    preferred_element_type=jnp.float32
                    ).astype(jnp.bfloat16)
        zpad = jnp.zeros((c_out, pad), jnp.bfloat16)
        vp = jnp.concatenate([zpad, v, zpad], axis=1)

        base_of = {0: vp * mask_l.astype(jnp.bfloat16), 1: vp,
                   2: vp * mask_r.astype(jnp.bfloat16)}

        acc = None
        for p in range(_KK):
            di, dj = divmod(p, _KSIZE)
            start = pad + (di - 1) * img_w + (dj - 1)
            vs = base_of[dj][:, start:start + s]
            term = wpred[p * c_out:(p + 1) * c_out, :] * vs
            acc = term if acc is None else acc + term
        o_ref[g] = acc.astype(o_ref.dtype)


def kernel(x, w1, gn_gamma, gn_beta, w2, b2, wv):
    bsz, c, h, w = x.shape
    s = h * w
    mid4 = 4 * w1.shape[0]
    c_out = 4 * wv.shape[0]
    dt = x.dtype

    x_flat = x.reshape(bsz, c, s)
    w1d, wvd, w2p = _expand_weights(w1, w2, b2, wv, dt)
    gam = gn_gamma.reshape(mid4, 1).astype(dt)
    bet = gn_beta.reshape(mid4, 1).astype(dt)

    g_batch = 4 if bsz % 4 == 0 else (2 if bsz % 2 == 0 else 1)
    out = pl.pallas_call(
        functools.partial(_fused_kernel, img_w=w, eps=_EPS, g_batch=g_batch),
        out_shape=jax.ShapeDtypeStruct((bsz, c_out, s), dt),
        grid=(bsz // g_batch,),
        in_specs=[
            pl.BlockSpec((g_batch, c, s), lambda b: (b, 0, 0)),
            pl.BlockSpec((mid4, c), lambda b: (0, 0)),
            pl.BlockSpec((mid4, 1), lambda b: (0, 0)),
            pl.BlockSpec((mid4, 1), lambda b: (0, 0)),
            pl.BlockSpec((_KK * c_out, 2 * mid4), lambda b: (0, 0)),
            pl.BlockSpec((c_out, c), lambda b: (0, 0)),
        ],
        out_specs=pl.BlockSpec((g_batch, c_out, s), lambda b: (b, 0, 0)),
        compiler_params=pltpu.CompilerParams(
            dimension_semantics=("parallel",)),
    )(x_flat, w1d, gam, bet, w2p, wvd)
    return out.reshape(bsz, c_out, h, w)


# speedup vs baseline: 1.1264x; 1.1104x over previous
import functools

import numpy as np
import jax
import jax.numpy as jnp
from jax.experimental import pallas as pl
from jax.experimental.pallas import tpu as pltpu

_KSIZE = 3
_KK = _KSIZE * _KSIZE
_EPS = 1e-5


def _rot_maps(k):
    i, j = np.meshgrid(np.arange(k), np.arange(k), indexing="ij")
    return np.stack([
        (i * k + j).ravel(),
        (j * k + (k - 1 - i)).ravel(),
        ((k - 1 - i) * k + (k - 1 - j)).ravel(),
        ((k - 1 - j) * k + i).ravel(),
    ], axis=0)


def _conv2_row_index(c_out):
    rot = _rot_maps(_KSIZE)
    taps, chans = np.meshgrid(np.arange(_KK), np.arange(c_out), indexing="ij")
    r = chans % 4
    q = (chans // 4) * _KK + rot[r, taps]
    return q.ravel(), r.ravel()


def _prep_kernel(w1r_ref, wvr_ref, w2_ref, b2_ref, qf_ref, rf_ref,
                 w1d_ref, wvd_ref, w2p_ref):
    def c4_expand(wr):
        o, n = wr.shape
        kk = jax.lax.broadcasted_iota(jnp.int32, (n, n), 0)
        nn = jax.lax.broadcasted_iota(jnp.int32, (n, n), 1)
        parts = []
        for s in range(4):
            src = nn - nn % 4 + (nn % 4 - s) % 4
            perm = (kk == src).astype(jnp.float32)
            parts.append(jnp.dot(wr[...], perm,
                                 preferred_element_type=jnp.float32))
        return jnp.stack(parts, axis=1).reshape(4 * o, n)

    w1d_ref[...] = c4_expand(w1r_ref).astype(w1d_ref.dtype)
    wvd_ref[...] = c4_expand(wvr_ref).astype(wvd_ref.dtype)

    nrow = qf_ref.shape[0]
    nq = w2_ref.shape[0]
    mid = w2_ref.shape[1]
    qcol = jax.lax.broadcasted_iota(jnp.int32, (nrow, nq), 1)
    onehot = (qcol == qf_ref[...]).astype(jnp.float32)
    rows4 = jnp.dot(onehot, w2_ref[...], preferred_element_type=jnp.float32)
    b2p = jnp.dot(onehot, b2_ref[...], preferred_element_type=jnp.float32)
    mm = jax.lax.broadcasted_iota(jnp.int32, (mid, 4 * mid), 0)
    nn = jax.lax.broadcasted_iota(jnp.int32, (mid, 4 * mid), 1)
    widen = (nn // 4 == mm).astype(jnp.float32)
    wide = jnp.dot(rows4, widen, preferred_element_type=jnp.float32)
    tcol = jax.lax.broadcasted_iota(jnp.int32, (nrow, 4 * mid), 1) % 4
    w2p = jnp.where(tcol == rf_ref[...], wide, 0.0)
    bcol = jax.lax.broadcasted_iota(jnp.int32, (nrow, 4 * mid), 1)
    bias_block = jnp.where(bcol == 0, b2p, 0.0)
    w2p_ref[...] = jnp.concatenate([w2p, bias_block],
                                   axis=1).astype(w2p_ref.dtype)


def _expand_weights(w1, w2, b2, wv, dt):
    mid4 = 4 * w1.shape[0]
    c_out = 4 * wv.shape[0]
    kc = _KK * c_out
    qf, rf = _conv2_row_index(c_out)
    return pl.pallas_call(
        _prep_kernel,
        out_shape=(jax.ShapeDtypeStruct((mid4, 4 * w1.shape[1]), dt),
                   jax.ShapeDtypeStruct((c_out, 4 * wv.shape[1]), dt),
                   jax.ShapeDtypeStruct((kc, 2 * mid4), jnp.bfloat16)),
    )(w1.reshape(w1.shape[0], -1), wv.reshape(wv.shape[0], -1),
      w2, b2.reshape(-1, 1),
      jnp.asarray(qf.reshape(-1, 1), jnp.int32),
      jnp.asarray(rf.reshape(-1, 1), jnp.int32))


def _fused_kernel(x_ref, w1_ref, gam_ref, bet_ref, w2p_ref, wv_ref,
                  o_ref, *, img_w, eps, g_batch):
    pad = 2 * img_w
    s = x_ref.shape[-1]
    mid4 = w1_ref.shape[0]
    c_out = wv_ref.shape[0]
    rr = jax.lax.broadcasted_iota(jnp.int32, (mid4, mid4), 0)
    cc = jax.lax.broadcasted_iota(jnp.int32, (mid4, mid4), 1)
    gmask = (rr // 4 == cc // 4).astype(jnp.float32)
    onesrow = (jax.lax.broadcasted_iota(jnp.int32, (mid4, s), 0) == 0
               ).astype(jnp.float32).astype(jnp.bfloat16)
    jcol = jax.lax.broadcasted_iota(jnp.int32, (1, s + 2 * pad), 1) % img_w
    mask_l = (jcol <= img_w - 2).astype(jnp.float32)
    mask_r = (jcol >= 1).astype(jnp.float32)
    n = jnp.float32(4 * s)

    ys = []
    stats = []
    for g in range(g_batch):
        y = jnp.dot(w1_ref[...], x_ref[g], preferred_element_type=jnp.float32)
        ys.append(y)
        stats.append(jnp.concatenate(
            [jnp.sum(y, axis=1, keepdims=True),
             jnp.sum(y * y, axis=1, keepdims=True)], axis=1))
    gagg = jnp.dot(gmask, jnp.concatenate(stats, axis=1),
                   preferred_element_type=jnp.float32) / n

    for g in range(g_batch):
        x = x_ref[g]
        y = ys[g]

        gmean = gagg[:, 2 * g:2 * g + 1]
        gm2 = gagg[:, 2 * g + 1:2 * g + 2]
        inv = jax.lax.rsqrt(gm2 - gmean * gmean + eps)
        scale = inv * gam_ref[...]
        shift = bet_ref[...] - gmean * scale
        yr = jnp.maximum(y * scale + shift, 0.0)

        yrb = jnp.concatenate([yr.astype(jnp.bfloat16), onesrow], axis=0)
        wpred = jnp.dot(w2p_ref[...], yrb,
                        preferred_element_type=jnp.float32
                        ).astype(jnp.bfloat16)

        v = jnp.dot(wv_ref[...].astype(jnp.bfloat16),
                    x.astype(jnp.bfloat16),
                    preferred_element_type=jnp.float32
                    ).astype(jnp.bfloat16)
        zpad = jnp.zeros((c_out, pad), jnp.bfloat16)
        vp = jnp.concatenate([zpad, v, zpad], axis=1)

        base_of = {0: vp * mask_l.astype(jnp.bfloat16), 1: vp,
                   2: vp * mask_r.astype(jnp.bfloat16)}

        acc = None
        for p in range(_KK):
            di, dj = divmod(p, _KSIZE)
            start = pad + (di - 1) * img_w + (dj - 1)
            vs = base_of[dj][:, start:start + s]
            term = wpred[p * c_out:(p + 1) * c_out, :] * vs
            acc = term if acc is None else acc + term
        o_ref[g] = acc.astype(o_ref.dtype)


def kernel(x, w1, gn_gamma, gn_beta, w2, b2, wv):
    bsz, c, h, w = x.shape
    s = h * w
    mid4 = 4 * w1.shape[0]
    c_out = 4 * wv.shape[0]
    dt = x.dtype

    x_flat = x.reshape(bsz, c, s)
    w1d, wvd, w2p = _expand_weights(w1, w2, b2, wv, dt)
    gam = gn_gamma.reshape(mid4, 1).astype(dt)
    bet = gn_beta.reshape(mid4, 1).astype(dt)

    g_batch = 4 if bsz % 4 == 0 else (2 if bsz % 2 == 0 else 1)
    out = pl.pallas_call(
        functools.partial(_fused_kernel, img_w=w, eps=_EPS, g_batch=g_batch),
        out_shape=jax.ShapeDtypeStruct((bsz, c_out, s), dt),
        grid=(bsz // g_batch,),
        in_specs=[
            pl.BlockSpec((g_batch, c, s), lambda b: (b, 0, 0)),
            pl.BlockSpec((mid4, c), lambda b: (0, 0)),
            pl.BlockSpec((mid4, 1), lambda b: (0, 0)),
            pl.BlockSpec((mid4, 1), lambda b: (0, 0)),
            pl.BlockSpec((_KK * c_out, 2 * mid4), lambda b: (0, 0)),
            pl.BlockSpec((c_out, c), lambda b: (0, 0)),
        ],
        out_specs=pl.BlockSpec((g_batch, c_out, s), lambda b: (b, 0, 0)),
        compiler_params=pltpu.CompilerParams(
            dimension_semantics=("parallel",)),
    )(x_flat, w1d, gam, bet, w2p, wvd)
    return out.reshape(bsz, c_out, h, w)


# speedup vs baseline: 1.1582x; 1.0283x over previous
import functools

import numpy as np
import jax
import jax.numpy as jnp
from jax.experimental import pallas as pl
from jax.experimental.pallas import tpu as pltpu

_KSIZE = 3
_KK = _KSIZE * _KSIZE
_EPS = 1e-5


def _rot_maps(k):
    i, j = np.meshgrid(np.arange(k), np.arange(k), indexing="ij")
    return np.stack([
        (i * k + j).ravel(),
        (j * k + (k - 1 - i)).ravel(),
        ((k - 1 - i) * k + (k - 1 - j)).ravel(),
        ((k - 1 - j) * k + i).ravel(),
    ], axis=0)


def _conv2_row_index(c_out):
    rot = _rot_maps(_KSIZE)
    taps, chans = np.meshgrid(np.arange(_KK), np.arange(c_out), indexing="ij")
    r = chans % 4
    q = (chans // 4) * _KK + rot[r, taps]
    return q.ravel(), r.ravel()


def _prep_kernel(w1r_ref, wvr_ref, w2_ref, b2_ref, qf_ref, rf_ref,
                 w1d_ref, wvd_ref, w2p_ref):
    def c4_expand(wr):
        o, n = wr.shape
        kk = jax.lax.broadcasted_iota(jnp.int32, (n, n), 0)
        nn = jax.lax.broadcasted_iota(jnp.int32, (n, n), 1)
        parts = []
        for s in range(4):
            src = nn - nn % 4 + (nn % 4 - s) % 4
            perm = (kk == src).astype(jnp.float32)
            parts.append(jnp.dot(wr[...], perm,
                                 preferred_element_type=jnp.float32))
        return jnp.stack(parts, axis=1).reshape(4 * o, n)

    w1d_ref[...] = c4_expand(w1r_ref).astype(w1d_ref.dtype)
    wvd_ref[...] = c4_expand(wvr_ref).astype(wvd_ref.dtype)

    nrow = qf_ref.shape[0]
    nq = w2_ref.shape[0]
    mid = w2_ref.shape[1]
    qcol = jax.lax.broadcasted_iota(jnp.int32, (nrow, nq), 1)
    onehot = (qcol == qf_ref[...]).astype(jnp.float32)
    rows4 = jnp.dot(onehot, w2_ref[...], preferred_element_type=jnp.float32)
    b2p = jnp.dot(onehot, b2_ref[...], preferred_element_type=jnp.float32)
    mm = jax.lax.broadcasted_iota(jnp.int32, (mid, 4 * mid), 0)
    nn = jax.lax.broadcasted_iota(jnp.int32, (mid, 4 * mid), 1)
    widen = (nn // 4 == mm).astype(jnp.float32)
    wide = jnp.dot(rows4, widen, preferred_element_type=jnp.float32)
    tcol = jax.lax.broadcasted_iota(jnp.int32, (nrow, 4 * mid), 1) % 4
    w2p = jnp.where(tcol == rf_ref[...], wide, 0.0)
    bcol = jax.lax.broadcasted_iota(jnp.int32, (nrow, 4 * mid), 1)
    bias_block = jnp.where(bcol == 0, b2p, 0.0)
    w2p_ref[...] = jnp.concatenate([w2p, bias_block],
                                   axis=1).astype(w2p_ref.dtype)


def _expand_weights(w1, w2, b2, wv, dt):
    mid4 = 4 * w1.shape[0]
    c_out = 4 * wv.shape[0]
    kc = _KK * c_out
    qf, rf = _conv2_row_index(c_out)
    return pl.pallas_call(
        _prep_kernel,
        out_shape=(jax.ShapeDtypeStruct((mid4, 4 * w1.shape[1]), dt),
                   jax.ShapeDtypeStruct((c_out, 4 * wv.shape[1]), dt),
                   jax.ShapeDtypeStruct((kc, 2 * mid4), jnp.bfloat16)),
    )(w1.reshape(w1.shape[0], -1), wv.reshape(wv.shape[0], -1),
      w2, b2.reshape(-1, 1),
      jnp.asarray(qf.reshape(-1, 1), jnp.int32),
      jnp.asarray(rf.reshape(-1, 1), jnp.int32))


def _fused_kernel(x_ref, w1_ref, gam_ref, bet_ref, w2p_ref, wv_ref,
                  o_ref, *, img_w, eps, g_batch):
    pad = 2 * img_w
    s = x_ref.shape[-1]
    mid4 = w1_ref.shape[0]
    c_out = wv_ref.shape[0]
    rr = jax.lax.broadcasted_iota(jnp.int32, (mid4, mid4), 0)
    cc = jax.lax.broadcasted_iota(jnp.int32, (mid4, mid4), 1)
    gmask = (rr // 4 == cc // 4).astype(jnp.float32)
    onesrow = (jax.lax.broadcasted_iota(jnp.int32, (mid4, s), 0) == 0
               ).astype(jnp.float32).astype(jnp.bfloat16)
    jcol = jax.lax.broadcasted_iota(jnp.int32, (1, s + 2 * pad), 1) % img_w
    mask_l = (jcol <= img_w - 2).astype(jnp.float32)
    mask_r = (jcol >= 1).astype(jnp.float32)
    n = jnp.float32(4 * s)

    ys = []
    stats = []
    for g in range(g_batch):
        y = jnp.dot(w1_ref[...], x_ref[g], preferred_element_type=jnp.float32)
        ys.append(y)
        stats.append(jnp.concatenate(
            [jnp.sum(y, axis=1, keepdims=True),
             jnp.sum(y * y, axis=1, keepdims=True)], axis=1))
    gagg = jnp.dot(gmask, jnp.concatenate(stats, axis=1),
                   preferred_element_type=jnp.float32) / n

    for g in range(g_batch):
        x = x_ref[g]
        y = ys[g]

        gmean = gagg[:, 2 * g:2 * g + 1]
        gm2 = gagg[:, 2 * g + 1:2 * g + 2]
        inv = jax.lax.rsqrt(gm2 - gmean * gmean + eps)
        scale = inv * gam_ref[...]
        shift = bet_ref[...] - gmean * scale
        yr = jnp.maximum(y * scale + shift, 0.0)

        yrb = jnp.concatenate([yr.astype(jnp.bfloat16), onesrow], axis=0)
        wpred = jnp.dot(w2p_ref[...], yrb,
                        preferred_element_type=jnp.float32
                        ).astype(jnp.bfloat16)

        v = jnp.dot(wv_ref[...].astype(jnp.bfloat16),
                    x.astype(jnp.bfloat16),
                    preferred_element_type=jnp.float32
                    ).astype(jnp.bfloat16)
        zpad = jnp.zeros((c_out, pad), jnp.bfloat16)
        vp = jnp.concatenate([zpad, v, zpad], axis=1)

        base_of = {0: vp * mask_l.astype(jnp.bfloat16), 1: vp,
                   2: vp * mask_r.astype(jnp.bfloat16)}

        acc = None
        for p in range(_KK):
            di, dj = divmod(p, _KSIZE)
            start = pad + (di - 1) * img_w + (dj - 1)
            vs = base_of[dj][:, start:start + s]
            term = wpred[p * c_out:(p + 1) * c_out, :] * vs
            acc = term if acc is None else acc + term
        o_ref[g] = acc.astype(o_ref.dtype)


def kernel(x, w1, gn_gamma, gn_beta, w2, b2, wv):
    bsz, c, h, w = x.shape
    s = h * w
    mid4 = 4 * w1.shape[0]
    c_out = 4 * wv.shape[0]
    dt = x.dtype

    x_flat = x.reshape(bsz, c, s)
    w1d, wvd, w2p = _expand_weights(w1, w2, b2, wv, dt)
    gam = gn_gamma.reshape(mid4, 1).astype(dt)
    bet = gn_beta.reshape(mid4, 1).astype(dt)

    g_batch = 8 if bsz % 8 == 0 else (2 if bsz % 2 == 0 else 1)
    out = pl.pallas_call(
        functools.partial(_fused_kernel, img_w=w, eps=_EPS, g_batch=g_batch),
        out_shape=jax.ShapeDtypeStruct((bsz, c_out, s), dt),
        grid=(bsz // g_batch,),
        in_specs=[
            pl.BlockSpec((g_batch, c, s), lambda b: (b, 0, 0)),
            pl.BlockSpec((mid4, c), lambda b: (0, 0)),
            pl.BlockSpec((mid4, 1), lambda b: (0, 0)),
            pl.BlockSpec((mid4, 1), lambda b: (0, 0)),
            pl.BlockSpec((_KK * c_out, 2 * mid4), lambda b: (0, 0)),
            pl.BlockSpec((c_out, c), lambda b: (0, 0)),
        ],
        out_specs=pl.BlockSpec((g_batch, c_out, s), lambda b: (b, 0, 0)),
        compiler_params=pltpu.CompilerParams(
            dimension_semantics=("parallel",)),
    )(x_flat, w1d, gam, bet, w2p, wvd)
    return out.reshape(bsz, c_out, h, w)


# speedup vs baseline: 1.1663x; 1.0070x over previous
import functools

import numpy as np
import jax
import jax.numpy as jnp
from jax.experimental import pallas as pl
from jax.experimental.pallas import tpu as pltpu

_KSIZE = 3
_KK = _KSIZE * _KSIZE
_EPS = 1e-5


def _rot_maps(k):
    i, j = np.meshgrid(np.arange(k), np.arange(k), indexing="ij")
    return np.stack([
        (i * k + j).ravel(),
        (j * k + (k - 1 - i)).ravel(),
        ((k - 1 - i) * k + (k - 1 - j)).ravel(),
        ((k - 1 - j) * k + i).ravel(),
    ], axis=0)


def _conv2_row_index(c_out):
    rot = _rot_maps(_KSIZE)
    taps, chans = np.meshgrid(np.arange(_KK), np.arange(c_out), indexing="ij")
    r = chans % 4
    q = (chans // 4) * _KK + rot[r, taps]
    return q.ravel(), r.ravel()


def _prep_kernel(w1r_ref, wvr_ref, w2_ref, b2_ref, qf_ref, rf_ref,
                 w1d_ref, wvd_ref, w2p_ref):
    def c4_expand(wr):
        o, n = wr.shape
        kk = jax.lax.broadcasted_iota(jnp.int32, (n, n), 0)
        nn = jax.lax.broadcasted_iota(jnp.int32, (n, n), 1)
        parts = []
        for s in range(4):
            src = nn - nn % 4 + (nn % 4 - s) % 4
            perm = (kk == src).astype(jnp.float32)
            parts.append(jnp.dot(wr[...], perm,
                                 preferred_element_type=jnp.float32))
        return jnp.stack(parts, axis=1).reshape(4 * o, n)

    w1d_ref[...] = c4_expand(w1r_ref).astype(w1d_ref.dtype)
    wvd_ref[...] = c4_expand(wvr_ref).astype(wvd_ref.dtype)

    nrow = qf_ref.shape[0]
    nq = w2_ref.shape[0]
    mid = w2_ref.shape[1]
    qcol = jax.lax.broadcasted_iota(jnp.int32, (nrow, nq), 1)
    onehot = (qcol == qf_ref[...]).astype(jnp.float32)
    rows4 = jnp.dot(onehot, w2_ref[...], preferred_element_type=jnp.float32)
    b2p = jnp.dot(onehot, b2_ref[...], preferred_element_type=jnp.float32)
    mm = jax.lax.broadcasted_iota(jnp.int32, (mid, 4 * mid), 0)
    nn = jax.lax.broadcasted_iota(jnp.int32, (mid, 4 * mid), 1)
    widen = (nn // 4 == mm).astype(jnp.float32)
    wide = jnp.dot(rows4, widen, preferred_element_type=jnp.float32)
    tcol = jax.lax.broadcasted_iota(jnp.int32, (nrow, 4 * mid), 1) % 4
    w2p = jnp.where(tcol == rf_ref[...], wide, 0.0)
    bcol = jax.lax.broadcasted_iota(jnp.int32, (nrow, 4 * mid), 1)
    bias_block = jnp.where(bcol == 0, b2p, 0.0)
    w2p_ref[...] = jnp.concatenate([w2p, bias_block],
                                   axis=1).astype(w2p_ref.dtype)


def _expand_weights(w1, w2, b2, wv, dt):
    mid4 = 4 * w1.shape[0]
    c_out = 4 * wv.shape[0]
    kc = _KK * c_out
    qf, rf = _conv2_row_index(c_out)
    return pl.pallas_call(
        _prep_kernel,
        out_shape=(jax.ShapeDtypeStruct((mid4, 4 * w1.shape[1]), dt),
                   jax.ShapeDtypeStruct((c_out, 4 * wv.shape[1]), dt),
                   jax.ShapeDtypeStruct((kc, 2 * mid4), jnp.bfloat16)),
    )(w1.reshape(w1.shape[0], -1), wv.reshape(wv.shape[0], -1),
      w2, b2.reshape(-1, 1),
      jnp.asarray(qf.reshape(-1, 1), jnp.int32),
      jnp.asarray(rf.reshape(-1, 1), jnp.int32))


def _fused_kernel(x_ref, w1_ref, gam_ref, bet_ref, w2p_ref, wv_ref,
                  o_ref, *, img_w, eps, g_batch):
    pad = 2 * img_w
    s = x_ref.shape[-1]
    mid4 = w1_ref.shape[0]
    c_out = wv_ref.shape[0]
    rr = jax.lax.broadcasted_iota(jnp.int32, (mid4, mid4), 0)
    cc = jax.lax.broadcasted_iota(jnp.int32, (mid4, mid4), 1)
    gmask = (rr // 4 == cc // 4).astype(jnp.float32)
    onesrow = (jax.lax.broadcasted_iota(jnp.int32, (mid4, s), 0) == 0
               ).astype(jnp.float32).astype(jnp.bfloat16)
    jcol = jax.lax.broadcasted_iota(jnp.int32, (1, s + 2 * pad), 1) % img_w
    mask_l = (jcol <= img_w - 2).astype(jnp.float32)
    mask_r = (jcol >= 1).astype(jnp.float32)
    n = jnp.float32(4 * s)

    ys = []
    stats = []
    for g in range(g_batch):
        y = jnp.dot(w1_ref[...], x_ref[g], preferred_element_type=jnp.float32)
        ys.append(y)
        stats.append(jnp.concatenate(
            [jnp.sum(y, axis=1, keepdims=True),
             jnp.sum(y * y, axis=1, keepdims=True)], axis=1))
    gagg = jnp.dot(gmask, jnp.concatenate(stats, axis=1),
                   preferred_element_type=jnp.float32) / n

    for g in range(g_batch):
        x = x_ref[g]
        y = ys[g]

        gmean = gagg[:, 2 * g:2 * g + 1]
        gm2 = gagg[:, 2 * g + 1:2 * g + 2]
        inv = jax.lax.rsqrt(gm2 - gmean * gmean + eps)
        scale = inv * gam_ref[...]
        shift = bet_ref[...] - gmean * scale
        yr = jnp.maximum(y * scale + shift, 0.0)

        yrb = jnp.concatenate([yr.astype(jnp.bfloat16), onesrow], axis=0)
        wpred = jnp.dot(w2p_ref[...], yrb,
                        preferred_element_type=jnp.float32
                        ).astype(jnp.bfloat16)

        v = jnp.dot(wv_ref[...].astype(jnp.bfloat16),
                    x.astype(jnp.bfloat16),
                    preferred_element_type=jnp.float32
                    ).astype(jnp.bfloat16)
        zpad = jnp.zeros((c_out, pad), jnp.bfloat16)
        vp = jnp.concatenate([zpad, v, zpad], axis=1)

        base_of = {0: vp * mask_l.astype(jnp.bfloat16), 1: vp,
                   2: vp * mask_r.astype(jnp.bfloat16)}

        acc = None
        for p in range(_KK):
            di, dj = divmod(p, _KSIZE)
            start = pad + (di - 1) * img_w + (dj - 1)
            vs = base_of[dj][:, start:start + s]
            term = wpred[p * c_out:(p + 1) * c_out, :] * vs
            acc = term if acc is None else acc + term
        o_ref[g] = acc.astype(o_ref.dtype)


def kernel(x, w1, gn_gamma, gn_beta, w2, b2, wv):
    bsz, c, h, w = x.shape
    s = h * w
    mid4 = 4 * w1.shape[0]
    c_out = 4 * wv.shape[0]
    dt = x.dtype

    x_flat = x.reshape(bsz, c, s)
    w1d, wvd, w2p = _expand_weights(w1, w2, b2, wv, dt)
    gam = gn_gamma.reshape(mid4, 1).astype(dt)
    bet = gn_beta.reshape(mid4, 1).astype(dt)

    g_batch = 16 if bsz % 16 == 0 else (2 if bsz % 2 == 0 else 1)
    out = pl.pallas_call(
        functools.partial(_fused_kernel, img_w=w, eps=_EPS, g_batch=g_batch),
        out_shape=jax.ShapeDtypeStruct((bsz, c_out, s), dt),
        grid=(bsz // g_batch,),
        in_specs=[
            pl.BlockSpec((g_batch, c, s), lambda b: (b, 0, 0)),
            pl.BlockSpec((mid4, c), lambda b: (0, 0)),
            pl.BlockSpec((mid4, 1), lambda b: (0, 0)),
            pl.BlockSpec((mid4, 1), lambda b: (0, 0)),
            pl.BlockSpec((_KK * c_out, 2 * mid4), lambda b: (0, 0)),
            pl.BlockSpec((c_out, c), lambda b: (0, 0)),
        ],
        out_specs=pl.BlockSpec((g_batch, c_out, s), lambda b: (b, 0, 0)),
        compiler_params=pltpu.CompilerParams(
            dimension_semantics=("parallel",)),
    )(x_flat, w1d, gam, bet, w2p, wvd)
    return out.reshape(bsz, c_out, h, w)
